# Initial kernel scaffold; baseline (speedup 1.0000x reference)
import jax
import jax.numpy as jnp
from jax.experimental import pallas as pl


def kernel(embeds, edge_index, qTrans, kTrans, vTrans, ln_scale, ln_bias):
    raise NotImplementedError("write your pallas kernel here")


# speedup vs baseline: 3.4981x; 3.4981x over previous
import jax
import jax.numpy as jnp
from jax import lax
from jax.experimental import pallas as pl
from jax.experimental.pallas import tpu as pltpu
from jax.experimental.pallas import tpu_sc as plsc

D = 128
HF = 64
NH = 8
NHC = 4
HD = 16
N_NODES = 10000
N_EDGES = 320000

NC = 2
NS = 16
E_PER_T = N_EDGES // NS
CHUNK = 160
N_CHUNKS = E_PER_T // CHUNK
N_PAD = 10240
NPT = N_PAD // NS


def _qkv_body(x_ref, wq_ref, wk_ref, wv_ref, q_ref, k_ref, v_ref):
    x = x_ref[...]
    for w_ref, o_ref in ((wq_ref, q_ref), (wk_ref, k_ref), (wv_ref, v_ref)):
        y = jnp.dot(x, w_ref[...], preferred_element_type=jnp.float32)
        o_ref[0] = y[:, :HF]
        o_ref[1] = y[:, HF:]


def _qkv(embeds, wq, wk, wv):
    bn = 1000
    grid = (N_NODES // bn,)
    out = jax.ShapeDtypeStruct((NC, N_NODES, HF), jnp.float32)
    return pl.pallas_call(
        _qkv_body,
        grid=grid,
        in_specs=[
            pl.BlockSpec((bn, D), lambda i: (i, 0)),
            pl.BlockSpec((D, D), lambda i: (0, 0)),
            pl.BlockSpec((D, D), lambda i: (0, 0)),
            pl.BlockSpec((D, D), lambda i: (0, 0)),
        ],
        out_specs=[
            pl.BlockSpec((NC, bn, HF), lambda i: (0, i, 0)),
            pl.BlockSpec((NC, bn, HF), lambda i: (0, i, 0)),
            pl.BlockSpec((NC, bn, HF), lambda i: (0, i, 0)),
        ],
        out_shape=[out, out, out],
    )(embeds, wq, wk, wv)


def _edge_body(q_hbm, k_hbm, v_hbm, rows_hbm, rc_hbm, cc_hbm, zrow_hbm,
               zattn_hbm, outp_hbm, attnp_hbm,
               rows_v, gq_v, gc_v, qv, kv, vv, wv, ea,
               out_acc, attn_acc, sem):
    c = lax.axis_index("c")
    s = lax.axis_index("s")

    pltpu.sync_copy(zrow_hbm.at[pl.ds(s * NPT, NPT)],
                    out_acc.at[pl.ds(s * NPT, NPT)])
    pltpu.sync_copy(zattn_hbm.at[pl.ds(s * NPT, NPT)],
                    attn_acc.at[pl.ds(s * NPT, NPT)])
    plsc.subcore_barrier()

    iota = lax.iota(jnp.int32, 16)
    perms = [iota ^ kk for kk in (8, 4, 2, 1)]
    gdn = lax.GatherDimensionNumbers(
        offset_dims=(), collapsed_slice_dims=(0,), start_index_map=(0,))

    def shuffle(x, p):
        return lax.gather(x, p[:, None], gdn, slice_sizes=(1,),
                          mode=lax.GatherScatterMode.PROMISE_IN_BOUNDS)

    def hsum(x):
        for p in perms:
            x = x + shuffle(x, p)
        return x

    def chunk(t, carry):
        base = s * E_PER_T + t * CHUNK
        gbase = c * N_EDGES + base
        pltpu.sync_copy(rows_hbm.at[pl.ds(base, CHUNK)], rows_v)
        pltpu.sync_copy(rc_hbm.at[pl.ds(gbase, CHUNK)], gq_v)
        pltpu.sync_copy(cc_hbm.at[pl.ds(gbase, CHUNK)], gc_v)
        pltpu.async_copy(q_hbm.at[gq_v], qv, sem).wait()
        pltpu.async_copy(k_hbm.at[gc_v], kv, sem).wait()
        pltpu.async_copy(v_hbm.at[gc_v], vv, sem).wait()

        def edge(e, carry2):
            acc = jnp.zeros((16,), jnp.float32)
            for h in range(NHC):
                qh = qv[e, pl.ds(h * HD, HD)]
                kh = kv[e, pl.ds(h * HD, HD)]
                bv = hsum(qh * kh)
                av = jnp.exp(jnp.minimum(jnp.maximum(bv, -10.0), 10.0))
                wv[e, pl.ds(h * HD, HD)] = av * vv[e, pl.ds(h * HD, HD)]
                acc = jnp.where(iota == h, av, acc)
            ea[e, :] = acc
            return carry2

        lax.fori_loop(0, CHUNK, edge, 0)
        pltpu.sync_copy(wv, out_acc.at[rows_v], add=True)
        pltpu.sync_copy(ea, attn_acc.at[rows_v], add=True)
        return carry

    lax.fori_loop(0, N_CHUNKS, chunk, 0)
    plsc.subcore_barrier()

    pltpu.sync_copy(out_acc.at[pl.ds(s * NPT, NPT)],
                    outp_hbm.at[pl.ds(c * N_PAD + s * NPT, NPT)])
    pltpu.sync_copy(attn_acc.at[pl.ds(s * NPT, NPT)],
                    attnp_hbm.at[pl.ds(c * N_PAD + s * NPT, NPT)])


def _edge_pass(q, k, v, rows, cols):
    mesh = plsc.VectorSubcoreMesh(core_axis_name="c", subcore_axis_name="s",
                                  num_cores=NC, num_subcores=NS)
    rc = jnp.concatenate([rows, rows + N_NODES])
    cc = jnp.concatenate([cols, cols + N_NODES])
    zrow = jnp.zeros((N_PAD, HF), jnp.float32)
    zattn = jnp.zeros((N_PAD, 16), jnp.float32)
    fn = pl.kernel(
        _edge_body,
        out_type=(jax.ShapeDtypeStruct((NC * N_PAD, HF), jnp.float32),
                  jax.ShapeDtypeStruct((NC * N_PAD, 16), jnp.float32)),
        mesh=mesh,
        scratch_types=[
            pltpu.VMEM((CHUNK,), jnp.int32),
            pltpu.VMEM((CHUNK,), jnp.int32),
            pltpu.VMEM((CHUNK,), jnp.int32),
            pltpu.VMEM((CHUNK, HF), jnp.float32),
            pltpu.VMEM((CHUNK, HF), jnp.float32),
            pltpu.VMEM((CHUNK, HF), jnp.float32),
            pltpu.VMEM((CHUNK, HF), jnp.float32),
            pltpu.VMEM((CHUNK, 16), jnp.float32),
            pltpu.VMEM_SHARED((N_PAD, HF), jnp.float32),
            pltpu.VMEM_SHARED((N_PAD, 16), jnp.float32),
            pltpu.SemaphoreType.DMA,
        ],
        compiler_params=pltpu.CompilerParams(use_tc_tiling_on_sc=False),
    )
    q2 = q.reshape(NC * N_NODES, HF)
    k2 = k.reshape(NC * N_NODES, HF)
    v2 = v.reshape(NC * N_NODES, HF)
    return fn(q2, k2, v2, rows, rc, cc, zrow, zattn)


def _final_body(outp_ref, attnp_ref, emb_ref, scale_ref, bias_ref, o_ref):
    raw = jnp.concatenate([outp_ref[0], outp_ref[1]], axis=-1)
    an = jnp.concatenate([attnp_ref[0, :, :NHC],
                          attnp_ref[1, :, :NHC]], axis=-1)
    r = lax.broadcasted_iota(jnp.int32, (NH, D), 0)
    col = lax.broadcasted_iota(jnp.int32, (NH, D), 1)
    expand = jnp.where((col // HD) == r, 1.0, 0.0).astype(jnp.float32)
    den = jnp.dot(an, expand, preferred_element_type=jnp.float32)
    x = raw / (den + 1e-8) + emb_ref[...]
    mean = jnp.mean(x, axis=-1, keepdims=True)
    var = jnp.mean((x - mean) ** 2, axis=-1, keepdims=True)
    o_ref[...] = ((x - mean) / jnp.sqrt(var + 1e-6)) * scale_ref[...] + bias_ref[...]


def _finalize(outp, attnp, embeds, ln_scale, ln_bias):
    bn = 1000
    grid = (N_NODES // bn,)
    outp3 = outp.reshape(NC, N_PAD, HF)[:, :N_NODES, :]
    attnp3 = attnp.reshape(NC, N_PAD, 16)[:, :N_NODES, :]
    return pl.pallas_call(
        _final_body,
        grid=grid,
        in_specs=[
            pl.BlockSpec((NC, bn, HF), lambda i: (0, i, 0)),
            pl.BlockSpec((NC, bn, 16), lambda i: (0, i, 0)),
            pl.BlockSpec((bn, D), lambda i: (i, 0)),
            pl.BlockSpec((1, D), lambda i: (0, 0)),
            pl.BlockSpec((1, D), lambda i: (0, 0)),
        ],
        out_specs=pl.BlockSpec((bn, D), lambda i: (i, 0)),
        out_shape=jax.ShapeDtypeStruct((N_NODES, D), jnp.float32),
    )(outp3, attnp3, embeds, ln_scale.reshape(1, D), ln_bias.reshape(1, D))


def kernel(embeds, edge_index, qTrans, kTrans, vTrans, ln_scale, ln_bias):
    rows = edge_index[0].astype(jnp.int32)
    cols = edge_index[1].astype(jnp.int32)
    q, k, v = _qkv(embeds, qTrans, kTrans, vTrans)
    outp, attnp = _edge_pass(q, k, v, rows, cols)
    return _finalize(outp, attnp, embeds, ln_scale, ln_bias)
